# baseline (speedup 1.0000x reference)
import functools

import jax
import jax.numpy as jnp
import numpy as np
from jax import lax
from jax.experimental import pallas as pl
from jax.experimental.pallas import tpu as pltpu
from jax.experimental.pallas import tpu_sc as plsc

_BATCH = 4096
_HIST = 50
_D = 128
_NC, _NS = 2, 16
_NW = _NC * _NS
_RPW = _BATCH // _NW
_LANES = 16
_SCALE = np.float32(np.sqrt(np.float32(_D)))
_UNROLL = 8
_HALF = _RPW // 2


def _scale_half(buf, lo):
    def rows(r8, carry):
        r = lo + r8 * _UNROLL
        for u in range(_UNROLL):
            for j in range(_D // _LANES):
                sl = pl.ds(j * _LANES, _LANES)
                buf[r + u, sl] = buf[r + u, sl] * _SCALE
        return carry

    lax.fori_loop(0, _HALF // _UNROLL, rows, 0)


@functools.partial(
    pl.kernel,
    out_type=jax.ShapeDtypeStruct((_HIST, _BATCH, _D), jnp.float32),
    mesh=plsc.VectorSubcoreMesh(core_axis_name="c", subcore_axis_name="s"),
    scratch_types=[
        pltpu.VMEM((_HIST, _RPW), jnp.int32),
        pltpu.VMEM((_RPW, _D), jnp.float32),
        pltpu.VMEM((_RPW, _D), jnp.float32),
        pltpu.VMEM((_RPW, _D), jnp.float32),
        pltpu.VMEM((_RPW, _D), jnp.float32),
        pltpu.SemaphoreType.DMA,
        pltpu.SemaphoreType.DMA,
        pltpu.SemaphoreType.DMA,
        pltpu.SemaphoreType.DMA,
        pltpu.SemaphoreType.DMA,
        pltpu.SemaphoreType.DMA,
        pltpu.SemaphoreType.DMA,
        pltpu.SemaphoreType.DMA,
    ],
)
def _sc_embed(idx_hbm, tab_hbm, out_hbm, idx_v, b0, b1, b2, b3,
              g0, g1, g2, g3, w0, w1, w2, w3):
    wid = lax.axis_index("s") * _NC + lax.axis_index("c")
    base = wid * _RPW
    pltpu.sync_copy(idx_hbm.at[wid], idx_v)

    bufs = (b0, b1, b2, b3)
    gsems = (g0, g1, g2, g3)
    wsems = (w0, w1, w2, w3)

    def fire_g(h, buf, sem):
        pltpu.async_copy(tab_hbm.at[idx_v.at[h]], buf, sem)

    def wait_g(buf, sem):
        pltpu.make_async_copy(tab_hbm.at[idx_v.at[0]], buf, sem).wait()

    def scale_write(buf, h, sem):
        _scale_half(buf, 0)
        pltpu.async_copy(buf.at[pl.ds(0, _HALF)],
                         out_hbm.at[h, pl.ds(base, _HALF)], sem)
        _scale_half(buf, _HALF)
        pltpu.async_copy(buf.at[pl.ds(_HALF, _HALF)],
                         out_hbm.at[h, pl.ds(base + _HALF, _HALF)], sem)

    def wait_w(buf, sem):
        pltpu.make_async_copy(tab_hbm.at[idx_v.at[0]], buf, sem).wait()

    def slot(h, h_next, refill):
        j = h % 4
        k = h_next % 4
        wait_g(bufs[j], gsems[j])
        scale_write(bufs[j], h, wsems[j])
        if refill:
            wait_w(bufs[k], wsems[k])
        fire_g(h_next, bufs[k], gsems[k])

    fire_g(0, b0, g0)
    fire_g(1, b1, g1)
    slot(0, 2, refill=False)
    slot(1, 3, refill=False)
    slot(2, 4, refill=True)
    slot(3, 5, refill=True)

    def step(i, carry):
        h = 4 * i
        slot_d(h, 0)
        slot_d(h + 1, 1)
        slot_d(h + 2, 2)
        slot_d(h + 3, 3)
        return carry

    def slot_d(h, j):
        k = (j + 2) % 4
        wait_g(bufs[j], gsems[j])
        scale_write(bufs[j], h, wsems[j])
        wait_w(bufs[k], wsems[k])
        fire_g(h + 2, bufs[k], gsems[k])

    lax.fori_loop(1, _HIST // 4, step, 0)

    wait_g(b0, g0)
    scale_write(b0, _HIST - 2, w0)
    wait_g(b1, g1)
    scale_write(b1, _HIST - 1, w1)
    wait_w(b2, w2)
    wait_w(b3, w3)
    wait_w(b0, w0)
    wait_w(b1, w1)


def kernel(x, input_embedding):
    xw = x.astype(jnp.int32).reshape(_NW, _RPW, _HIST).transpose(0, 2, 1)
    out = _sc_embed(xw, input_embedding)
    return jnp.transpose(out, (1, 0, 2))


# speedup vs baseline: 1.0303x; 1.0303x over previous
import functools

import jax
import jax.numpy as jnp
import numpy as np
from jax import lax
from jax.experimental import pallas as pl
from jax.experimental.pallas import tpu as pltpu
from jax.experimental.pallas import tpu_sc as plsc

_BATCH = 4096
_HIST = 50
_D = 128
_NC, _NS = 2, 16
_NW = _NC * _NS
_RPW = _BATCH // _NW
_LANES = 16
_SCALE = np.float32(np.sqrt(np.float32(_D)))
_UNROLL = 8


def _scale_rows(buf):
    def rows(r4, carry):
        r = r4 * _UNROLL
        for u in range(_UNROLL):
            for j in range(_D // _LANES):
                sl = pl.ds(j * _LANES, _LANES)
                buf[r + u, sl] = buf[r + u, sl] * _SCALE
        return carry

    lax.fori_loop(0, _RPW // _UNROLL, rows, 0)


@functools.partial(
    pl.kernel,
    out_type=jax.ShapeDtypeStruct((_HIST, _BATCH, _D), jnp.float32),
    mesh=plsc.VectorSubcoreMesh(core_axis_name="c", subcore_axis_name="s"),
    scratch_types=[
        pltpu.VMEM((_HIST, _RPW), jnp.int32),
        pltpu.VMEM((_RPW, _D), jnp.float32),
        pltpu.VMEM((_RPW, _D), jnp.float32),
        pltpu.VMEM((_RPW, _D), jnp.float32),
        pltpu.VMEM((_RPW, _D), jnp.float32),
        pltpu.SemaphoreType.DMA,
        pltpu.SemaphoreType.DMA,
        pltpu.SemaphoreType.DMA,
        pltpu.SemaphoreType.DMA,
        pltpu.SemaphoreType.DMA,
        pltpu.SemaphoreType.DMA,
        pltpu.SemaphoreType.DMA,
        pltpu.SemaphoreType.DMA,
    ],
)
def _sc_embed(idx_hbm, tab_hbm, out_hbm, idx_v, b0, b1, b2, b3,
              g0, g1, g2, g3, w0, w1, w2, w3):
    wid = lax.axis_index("s") * _NC + lax.axis_index("c")
    base = wid * _RPW
    pltpu.sync_copy(idx_hbm.at[wid], idx_v)

    bufs = (b0, b1, b2, b3)
    gsems = (g0, g1, g2, g3)
    wsems = (w0, w1, w2, w3)

    def fire_g(h, buf, sem):
        pltpu.async_copy(tab_hbm.at[idx_v.at[h]], buf, sem)

    def wait_g(buf, sem):
        pltpu.make_async_copy(tab_hbm.at[idx_v.at[0]], buf, sem).wait()

    def fire_w(buf, h, sem):
        pltpu.async_copy(buf, out_hbm.at[h, pl.ds(base, _RPW)], sem)

    def wait_w(buf, sem):
        pltpu.make_async_copy(tab_hbm.at[idx_v.at[0]], buf, sem).wait()

    def slot(h, h_next, refill):
        j = h % 4
        k = h_next % 4
        wait_g(bufs[j], gsems[j])
        _scale_rows(bufs[j])
        fire_w(bufs[j], h, wsems[j])
        if refill:
            wait_w(bufs[k], wsems[k])
        fire_g(h_next, bufs[k], gsems[k])

    fire_g(0, b0, g0)
    fire_g(1, b1, g1)
    slot(0, 2, refill=False)
    slot(1, 3, refill=False)
    slot(2, 4, refill=True)
    slot(3, 5, refill=True)

    def step(i, carry):
        h = 4 * i
        slot_d(h, 0)
        slot_d(h + 1, 1)
        slot_d(h + 2, 2)
        slot_d(h + 3, 3)
        return carry

    def slot_d(h, j):
        k = (j + 2) % 4
        wait_g(bufs[j], gsems[j])
        _scale_rows(bufs[j])
        fire_w(bufs[j], h, wsems[j])
        wait_w(bufs[k], wsems[k])
        fire_g(h + 2, bufs[k], gsems[k])

    lax.fori_loop(1, _HIST // 4, step, 0)

    wait_g(b0, g0)
    _scale_rows(b0)
    fire_w(b0, _HIST - 2, w0)
    wait_g(b1, g1)
    _scale_rows(b1)
    fire_w(b1, _HIST - 1, w1)
    wait_w(b2, w2)
    wait_w(b3, w3)
    wait_w(b0, w0)
    wait_w(b1, w1)


def kernel(x, input_embedding):
    xw = x.astype(jnp.int32).reshape(_NW, _RPW, _HIST).transpose(0, 2, 1)
    out = _sc_embed(xw, input_embedding)
    return jnp.transpose(out, (1, 0, 2))


# speedup vs baseline: 1.0330x; 1.0025x over previous
import functools

import jax
import jax.numpy as jnp
import numpy as np
from jax import lax
from jax.experimental import pallas as pl
from jax.experimental.pallas import tpu as pltpu
from jax.experimental.pallas import tpu_sc as plsc

_BATCH = 4096
_HIST = 50
_D = 128
_NC, _NS = 2, 16
_NW = _NC * _NS
_RPW = _BATCH // _NW
_LANES = 16
_SCALE = np.float32(np.sqrt(np.float32(_D)))
_UNROLL = 4
_NB = 5


def _scale_rows(buf):
    def rows(r4, carry):
        r = r4 * _UNROLL
        for u in range(_UNROLL):
            for j in range(_D // _LANES):
                sl = pl.ds(j * _LANES, _LANES)
                buf[r + u, sl] = buf[r + u, sl] * _SCALE
        return carry

    lax.fori_loop(0, _RPW // _UNROLL, rows, 0)


@functools.partial(
    pl.kernel,
    out_type=jax.ShapeDtypeStruct((_HIST, _BATCH, _D), jnp.float32),
    mesh=plsc.VectorSubcoreMesh(core_axis_name="c", subcore_axis_name="s"),
    scratch_types=[
        pltpu.VMEM((_HIST * _RPW,), jnp.int32),
    ] + [pltpu.VMEM((_RPW, _D), jnp.float32)] * _NB
      + [pltpu.SemaphoreType.DMA] * (2 * _NB),
)
def _sc_embed(idx_hbm, tab_hbm, out_hbm, idx_v, *bufs_sems):
    bufs = bufs_sems[:_NB]
    gsems = bufs_sems[_NB:2 * _NB]
    wsems = bufs_sems[2 * _NB:]
    wid = lax.axis_index("s") * _NC + lax.axis_index("c")
    base = wid * _RPW
    pltpu.sync_copy(idx_hbm.at[wid], idx_v)

    def fire_g(h, buf, sem):
        pltpu.async_copy(tab_hbm.at[idx_v.at[pl.ds(h * _RPW, _RPW)]], buf, sem)

    def wait_g(buf, sem):
        pltpu.make_async_copy(tab_hbm.at[idx_v.at[pl.ds(0, _RPW)]],
                              buf, sem).wait()

    def fire_w(buf, h, sem):
        pltpu.async_copy(buf, out_hbm.at[h, pl.ds(base, _RPW)], sem)

    def wait_w(buf, sem):
        pltpu.make_async_copy(tab_hbm.at[idx_v.at[pl.ds(0, _RPW)]],
                              buf, sem).wait()

    def slot(h, j, refill, fire):
        k = (j + 2) % _NB
        wait_g(bufs[j], gsems[j])
        _scale_rows(bufs[j])
        fire_w(bufs[j], h, wsems[j])
        if fire:
            if refill:
                wait_w(bufs[k], wsems[k])
            fire_g(h + 2, bufs[k], gsems[k])

    fire_g(0, bufs[0], gsems[0])
    fire_g(1, bufs[1], gsems[1])
    slot(0, 0, refill=False, fire=True)
    slot(1, 1, refill=False, fire=True)
    slot(2, 2, refill=False, fire=True)
    slot(3, 3, refill=True, fire=True)
    slot(4, 4, refill=True, fire=True)

    def step(i, carry):
        h = _NB * i
        for j in range(_NB):
            slot(h + j, j, refill=True, fire=True)
        return carry

    lax.fori_loop(1, _HIST // _NB - 1, step, 0)

    slot(45, 0, refill=True, fire=True)
    slot(46, 1, refill=True, fire=True)
    slot(47, 2, refill=True, fire=True)
    slot(48, 3, refill=False, fire=False)
    slot(49, 4, refill=False, fire=False)
    for j in range(_NB):
        wait_w(bufs[j], wsems[j])


def kernel(x, input_embedding):
    xw = (x.astype(jnp.int32).reshape(_NW, _RPW, _HIST).transpose(0, 2, 1)
          .reshape(_NW, _HIST * _RPW))
    out = _sc_embed(xw, input_embedding)
    return jnp.transpose(out, (1, 0, 2))


# speedup vs baseline: 1.0694x; 1.0353x over previous
import functools

import jax
import jax.numpy as jnp
import numpy as np
from jax import lax
from jax.experimental import pallas as pl
from jax.experimental.pallas import tpu as pltpu
from jax.experimental.pallas import tpu_sc as plsc

_BATCH = 4096
_HIST = 50
_D = 128
_NC, _NS = 2, 16
_NW = _NC * _NS
_RPW = _BATCH // _NW
_LANES = 16
_SCALE = np.float32(np.sqrt(np.float32(_D)))
_UNROLL = 4


def _scale_rows(buf):
    def rows(r4, carry):
        r = r4 * _UNROLL
        for u in range(_UNROLL):
            for j in range(_D // _LANES):
                sl = pl.ds(j * _LANES, _LANES)
                buf[r + u, sl] = buf[r + u, sl] * _SCALE
        return carry

    lax.fori_loop(0, _RPW // _UNROLL, rows, 0)


@functools.partial(
    pl.kernel,
    out_type=jax.ShapeDtypeStruct((_HIST, _BATCH, _D), jnp.float32),
    mesh=plsc.VectorSubcoreMesh(core_axis_name="c", subcore_axis_name="s"),
    scratch_types=[
        pltpu.VMEM((_HIST, _RPW), jnp.int32),
        pltpu.VMEM((_RPW, _D), jnp.float32),
        pltpu.VMEM((_RPW, _D), jnp.float32),
        pltpu.VMEM((_RPW, _D), jnp.float32),
        pltpu.VMEM((_RPW, _D), jnp.float32),
        pltpu.SemaphoreType.DMA,
        pltpu.SemaphoreType.DMA,
        pltpu.SemaphoreType.DMA,
        pltpu.SemaphoreType.DMA,
        pltpu.SemaphoreType.DMA,
        pltpu.SemaphoreType.DMA,
        pltpu.SemaphoreType.DMA,
        pltpu.SemaphoreType.DMA,
    ],
)
def _sc_embed(idx_hbm, tab_hbm, out_hbm, idx_v, b0, b1, b2, b3,
              g0, g1, g2, g3, w0, w1, w2, w3):
    wid = lax.axis_index("s") * _NC + lax.axis_index("c")
    base = wid * _RPW
    pltpu.sync_copy(idx_hbm.at[wid], idx_v)

    bufs = (b0, b1, b2, b3)
    gsems = (g0, g1, g2, g3)
    wsems = (w0, w1, w2, w3)

    def fire_g(h, buf, sem):
        pltpu.async_copy(tab_hbm.at[idx_v.at[h]], buf, sem)

    def wait_g(buf, sem):
        pltpu.make_async_copy(tab_hbm.at[idx_v.at[0]], buf, sem).wait()

    def fire_w(buf, h, sem):
        pltpu.async_copy(buf, out_hbm.at[h, pl.ds(base, _RPW)], sem)

    def wait_w(buf, sem):
        pltpu.make_async_copy(tab_hbm.at[idx_v.at[0]], buf, sem).wait()

    def slot(h, h_next, refill):
        j = h % 4
        k = h_next % 4
        wait_g(bufs[j], gsems[j])
        if refill:
            wait_w(bufs[k], wsems[k])
        fire_g(h_next, bufs[k], gsems[k])
        _scale_rows(bufs[j])
        fire_w(bufs[j], h, wsems[j])

    fire_g(0, b0, g0)
    fire_g(1, b1, g1)
    slot(0, 2, refill=False)
    slot(1, 3, refill=False)
    slot(2, 4, refill=True)
    slot(3, 5, refill=True)

    def step(i, carry):
        h = 4 * i
        slot_d(h, 0)
        slot_d(h + 1, 1)
        slot_d(h + 2, 2)
        slot_d(h + 3, 3)
        return carry

    def slot_d(h, j):
        k = (j + 2) % 4
        wait_g(bufs[j], gsems[j])
        wait_w(bufs[k], wsems[k])
        fire_g(h + 2, bufs[k], gsems[k])
        _scale_rows(bufs[j])
        fire_w(bufs[j], h, wsems[j])

    lax.fori_loop(1, _HIST // 4, step, 0)

    wait_g(b0, g0)
    _scale_rows(b0)
    fire_w(b0, _HIST - 2, w0)
    wait_g(b1, g1)
    _scale_rows(b1)
    fire_w(b1, _HIST - 1, w1)
    wait_w(b2, w2)
    wait_w(b3, w3)
    wait_w(b0, w0)
    wait_w(b1, w1)


def kernel(x, input_embedding):
    xw = x.astype(jnp.int32).reshape(_NW, _RPW, _HIST).transpose(0, 2, 1)
    out = _sc_embed(xw, input_embedding)
    return jnp.transpose(out, (1, 0, 2))
